# Initial kernel scaffold; baseline (speedup 1.0000x reference)
import functools

import jax
import jax.numpy as jnp
from jax import lax
from jax.experimental import pallas as pl
from jax.experimental.pallas import tpu as pltpu
from jax.experimental.pallas import tpu_sc as plsc

N_NODES = 10000
N_EDGES = 320000
D_IN = 128
HID = 256
N_CLS = 40
W_PAD = 48

NC = 2
NS = 16
NW = NC * NS
EROW = 128
ROWS_E = N_EDGES // EROW
TILE_ROWS = N_NODES // NS


def _make_prop(width):
  mesh = plsc.VectorSubcoreMesh(core_axis_name="c", subcore_axis_name="s")

  @functools.partial(
      pl.kernel,
      out_type=jax.ShapeDtypeStruct((NC, N_NODES, width), jnp.float32),
      mesh=mesh,
      scratch_types=[
          pltpu.VMEM((1, EROW), jnp.int32),
          pltpu.VMEM((1, EROW), jnp.int32),
          pltpu.VMEM((EROW, width), jnp.float32),
          pltpu.VMEM_SHARED((N_NODES, width), jnp.float32),
          pltpu.SemaphoreType.DMA,
      ],
  )
  def prop(x_hbm, src_hbm, dst_hbm, zeros_hbm, out_hbm, idx_s, idx_d, rows,
           acc, sem):
    c = lax.axis_index("c")
    s = lax.axis_index("s")
    wid = s * NC + c
    pltpu.sync_copy(zeros_hbm.at[pl.ds(s * TILE_ROWS, TILE_ROWS)],
                    acc.at[pl.ds(s * TILE_ROWS, TILE_ROWS)])
    plsc.subcore_barrier()
    start = wid * ROWS_E // NW
    stop = (wid + 1) * ROWS_E // NW

    def body(r, carry):
      pltpu.sync_copy(src_hbm.at[pl.ds(r, 1)], idx_s)
      pltpu.sync_copy(dst_hbm.at[pl.ds(r, 1)], idx_d)
      pltpu.async_copy(x_hbm.at[idx_s], rows, sem).wait()
      pltpu.sync_copy(rows, acc.at[idx_d], add=True)
      return carry

    lax.fori_loop(start, stop, body, 0)
    plsc.subcore_barrier()
    pltpu.sync_copy(acc.at[pl.ds(s * TILE_ROWS, TILE_ROWS)],
                    out_hbm.at[c, pl.ds(s * TILE_ROWS, TILE_ROWS)])

  return prop


_prop_deg = _make_prop(1)
_prop_l1 = _make_prop(D_IN)
_prop_l2 = _make_prop(W_PAD)


def _dinv_body(p_ref, o_ref):
  deg = jnp.sum(p_ref[...], axis=0, keepdims=True)
  o_ref[...] = lax.rsqrt(jnp.maximum(deg, 1.0))


def _pre_body(feat_ref, dinv_ref, w1a_ref, b1_ref, xs_ref, z1_ref):
  f = feat_ref[...]
  xs_ref[...] = f * dinv_ref[...]
  z1_ref[...] = (
      jnp.dot(f, w1a_ref[...], preferred_element_type=jnp.float32)
      + b1_ref[...])


def _mid_body(z1_ref, a0_ref, a1_ref, dinv_ref, w1b_ref, w2_ref, b2_ref,
              y2a_ref, y2b_ref):
  d = dinv_ref[...]
  agg = (a0_ref[...] + a1_ref[...]) * d
  h = jnp.maximum(
      z1_ref[...]
      - jnp.dot(agg, w1b_ref[...], preferred_element_type=jnp.float32), 0.0)
  y = jnp.dot(h, w2_ref[...], preferred_element_type=jnp.float32)
  y2a_ref[...] = y[:, :N_CLS] + b2_ref[...]
  yb = y[:, N_CLS:] * d
  pad = jnp.zeros((yb.shape[0], W_PAD - N_CLS), jnp.float32)
  y2b_ref[...] = jnp.concatenate([yb, pad], axis=1)


def _fin_body(y2a_ref, q0_ref, q1_ref, dinv_ref, o_ref):
  q = (q0_ref[...] + q1_ref[...])[:, :N_CLS]
  o = y2a_ref[...] - q * dinv_ref[...]
  m = jnp.max(o, axis=1, keepdims=True)
  lse = jnp.log(jnp.sum(jnp.exp(o - m), axis=1, keepdims=True)) + m
  o_ref[...] = o - lse


_R = 1000
_GRID = (N_NODES // _R,)


def _rows(w):
  return pl.BlockSpec((_R, w), lambda i: (i, 0))


def _full(a, b):
  return pl.BlockSpec((a, b), lambda i: (0, 0))


_dinv_call = pl.pallas_call(
    _dinv_body,
    out_shape=jax.ShapeDtypeStruct((1, N_NODES), jnp.float32),
)

_pre_call = pl.pallas_call(
    _pre_body,
    grid=_GRID,
    in_specs=[_rows(D_IN), _rows(1), _full(D_IN, HID), _full(1, HID)],
    out_specs=[_rows(D_IN), _rows(HID)],
    out_shape=[
        jax.ShapeDtypeStruct((N_NODES, D_IN), jnp.float32),
        jax.ShapeDtypeStruct((N_NODES, HID), jnp.float32),
    ],
)

_mid_call = pl.pallas_call(
    _mid_body,
    grid=_GRID,
    in_specs=[
        _rows(HID), _rows(D_IN), _rows(D_IN), _rows(1),
        _full(D_IN, HID), _full(HID, 2 * N_CLS), _full(1, N_CLS),
    ],
    out_specs=[_rows(N_CLS), _rows(W_PAD)],
    out_shape=[
        jax.ShapeDtypeStruct((N_NODES, N_CLS), jnp.float32),
        jax.ShapeDtypeStruct((N_NODES, W_PAD), jnp.float32),
    ],
)

_fin_call = pl.pallas_call(
    _fin_body,
    grid=_GRID,
    in_specs=[_rows(N_CLS), _rows(W_PAD), _rows(W_PAD), _rows(1)],
    out_specs=_rows(N_CLS),
    out_shape=jax.ShapeDtypeStruct((N_NODES, N_CLS), jnp.float32),
)


@jax.jit
def kernel(feat, edge_index, W1, b1, W2, b2):
  src = edge_index[0].reshape(ROWS_E, EROW)
  dst = edge_index[1].reshape(ROWS_E, EROW)
  ones_n = jnp.ones((N_NODES, 1), jnp.float32)
  zeros1 = jnp.zeros((N_NODES, 1), jnp.float32)
  zeros128 = jnp.zeros((N_NODES, D_IN), jnp.float32)
  zeros48 = jnp.zeros((N_NODES, W_PAD), jnp.float32)

  degp = _prop_deg(ones_n, src, dst, zeros1)
  dinv = _dinv_call(degp.reshape(NC, N_NODES)).reshape(N_NODES, 1)
  xs, z1 = _pre_call(feat, dinv, W1[:D_IN], b1.reshape(1, HID))
  a = _prop_l1(xs, src, dst, zeros128)
  w2cat = jnp.concatenate([W2[:HID], W2[HID:]], axis=1)
  y2a, y2b = _mid_call(z1, a[0], a[1], dinv, W1[D_IN:], w2cat,
                       b2.reshape(1, N_CLS))
  q = _prop_l2(y2b, src, dst, zeros48)
  return _fin_call(y2a, q[0], q[1], dinv)


# speedup vs baseline: 5.7662x; 5.7662x over previous
import functools

import jax
import jax.numpy as jnp
from jax import lax
from jax.experimental import pallas as pl
from jax.experimental.pallas import tpu as pltpu
from jax.experimental.pallas import tpu_sc as plsc

N_NODES = 10000
N_EDGES = 320000
D_IN = 128
HID = 256
N_CLS = 40
W_PAD = 48

N_PAD = 10240

NC = 2
NS = 16
NW = NC * NS
EROW = 128
ROWS_E = N_EDGES // EROW
TILE_ROWS = N_PAD // NS


def _make_prop(width):
  mesh = plsc.VectorSubcoreMesh(
      core_axis_name="c", subcore_axis_name="s", num_cores=NC, num_subcores=NS)

  @functools.partial(
      pl.kernel,
      out_type=jax.ShapeDtypeStruct((NC, N_PAD, width), jnp.float32),
      mesh=mesh,
      scratch_types=[
          pltpu.VMEM((EROW,), jnp.int32),
          pltpu.VMEM((EROW,), jnp.int32),
          pltpu.VMEM((EROW, width), jnp.float32),
          pltpu.VMEM_SHARED((N_PAD, width), jnp.float32),
          pltpu.SemaphoreType.DMA,
      ],
      compiler_params=pltpu.CompilerParams(use_tc_tiling_on_sc=False),
  )
  def prop(x_hbm, src_hbm, dst_hbm, zeros_hbm, out_hbm, idx_s, idx_d, rows,
           acc, sem):
    c = lax.axis_index("c")
    s = lax.axis_index("s")
    wid = s * NC + c
    pltpu.sync_copy(zeros_hbm.at[pl.ds(s * TILE_ROWS, TILE_ROWS)],
                    acc.at[pl.ds(s * TILE_ROWS, TILE_ROWS)])
    plsc.subcore_barrier()
    start = wid * ROWS_E // NW
    stop = (wid + 1) * ROWS_E // NW

    def body(r, carry):
      pltpu.sync_copy(src_hbm.at[pl.ds(r * EROW, EROW)], idx_s)
      pltpu.sync_copy(dst_hbm.at[pl.ds(r * EROW, EROW)], idx_d)
      pltpu.async_copy(x_hbm.at[idx_s], rows, sem).wait()
      pltpu.sync_copy(rows, acc.at[idx_d], add=True)
      return carry

    lax.fori_loop(start, stop, body, 0)
    plsc.subcore_barrier()
    pltpu.sync_copy(acc.at[pl.ds(s * TILE_ROWS, TILE_ROWS)],
                    out_hbm.at[c, pl.ds(s * TILE_ROWS, TILE_ROWS)])

  return prop


_prop = functools.lru_cache(maxsize=None)(_make_prop)


def _dinv_body(p_ref, o_ref):
  deg = jnp.sum(p_ref[...], axis=0, keepdims=True)
  o_ref[...] = lax.rsqrt(jnp.maximum(deg, 1.0))


def _pre_body(feat_ref, dinv_ref, w1a_ref, b1_ref, xs_ref, z1_ref):
  f = feat_ref[...]
  xs_ref[...] = f * dinv_ref[...]
  z1_ref[...] = (
      jnp.dot(f, w1a_ref[...], preferred_element_type=jnp.float32)
      + b1_ref[...])


def _mid_body(z1_ref, a0_ref, a1_ref, dinv_ref, w1b_ref, w2_ref, b2_ref,
              y2a_ref, y2b_ref):
  d = dinv_ref[...]
  agg = (a0_ref[...] + a1_ref[...]) * d
  h = jnp.maximum(
      z1_ref[...]
      - jnp.dot(agg, w1b_ref[...], preferred_element_type=jnp.float32), 0.0)
  y = jnp.dot(h, w2_ref[...], preferred_element_type=jnp.float32)
  y2a_ref[...] = y[:, :N_CLS] + b2_ref[...]
  yb = y[:, N_CLS:] * d
  pad = jnp.zeros((yb.shape[0], W_PAD - N_CLS), jnp.float32)
  y2b_ref[...] = jnp.concatenate([yb, pad], axis=1)


def _fin_body(y2a_ref, q0_ref, q1_ref, dinv_ref, o_ref):
  q = (q0_ref[...] + q1_ref[...])[:, :N_CLS]
  o = y2a_ref[...] - q * dinv_ref[...]
  m = jnp.max(o, axis=1, keepdims=True)
  lse = jnp.log(jnp.sum(jnp.exp(o - m), axis=1, keepdims=True)) + m
  o_ref[...] = o - lse


_R = 1000
_GRID = (N_NODES // _R,)


def _rows(w):
  return pl.BlockSpec((_R, w), lambda i: (i, 0))


def _full(a, b):
  return pl.BlockSpec((a, b), lambda i: (0, 0))


_dinv_call = pl.pallas_call(
    _dinv_body,
    out_shape=jax.ShapeDtypeStruct((1, N_NODES), jnp.float32),
)

_pre_call = pl.pallas_call(
    _pre_body,
    grid=_GRID,
    in_specs=[_rows(D_IN), _rows(1), _full(D_IN, HID), _full(1, HID)],
    out_specs=[_rows(D_IN), _rows(HID)],
    out_shape=[
        jax.ShapeDtypeStruct((N_NODES, D_IN), jnp.float32),
        jax.ShapeDtypeStruct((N_NODES, HID), jnp.float32),
    ],
)

_mid_call = pl.pallas_call(
    _mid_body,
    grid=_GRID,
    in_specs=[
        _rows(HID), _rows(D_IN), _rows(D_IN), _rows(1),
        _full(D_IN, HID), _full(HID, 2 * N_CLS), _full(1, N_CLS),
    ],
    out_specs=[_rows(N_CLS), _rows(W_PAD)],
    out_shape=[
        jax.ShapeDtypeStruct((N_NODES, N_CLS), jnp.float32),
        jax.ShapeDtypeStruct((N_NODES, W_PAD), jnp.float32),
    ],
)

_fin_call = pl.pallas_call(
    _fin_body,
    grid=_GRID,
    in_specs=[_rows(N_CLS), _rows(W_PAD), _rows(W_PAD), _rows(1)],
    out_specs=_rows(N_CLS),
    out_shape=jax.ShapeDtypeStruct((N_NODES, N_CLS), jnp.float32),
)


@jax.jit
def kernel(feat, edge_index, W1, b1, W2, b2):
  src = edge_index[0]
  dst = edge_index[1]
  ones_n = jnp.ones((N_NODES, 8), jnp.float32)
  zeros1 = jnp.zeros((N_PAD, 8), jnp.float32)
  zeros128 = jnp.zeros((N_PAD, D_IN), jnp.float32)
  zeros48 = jnp.zeros((N_PAD, W_PAD), jnp.float32)

  degp = _prop(8)(ones_n, src, dst, zeros1)[:, :N_NODES, 0]
  dinv = _dinv_call(degp).reshape(N_NODES, 1)
  xs, z1 = _pre_call(feat, dinv, W1[:D_IN], b1.reshape(1, HID))
  a = _prop(D_IN)(xs, src, dst, zeros128)[:, :N_NODES]
  w2cat = jnp.concatenate([W2[:HID], W2[HID:]], axis=1)
  y2a, y2b = _mid_call(z1, a[0], a[1], dinv, W1[D_IN:], w2cat,
                       b2.reshape(1, N_CLS))
  q = _prop(W_PAD)(y2b, src, dst, zeros48)[:, :N_NODES]
  return _fin_call(y2a, q[0], q[1], dinv)
